# baseline (speedup 1.0000x reference)
import functools

import jax
import jax.numpy as jnp
from jax import lax
from jax.experimental import pallas as pl
from jax.experimental.pallas import tpu as pltpu
from jax.experimental.pallas import tpu_sc as plsc

_N = 10000
_E = 320000
_D = 128
_CP = 16
_COORDS_RANGE = 10.0

_NC = 2
_NS = 16
_NW = _NC * _NS
_EPW = _E // _NW

_CH = 200
_NCHUNK = _EPW // _CH
_NGRP = (_CH + 15) // 16

_CHS = 200
_NCHUNKS = _EPW // _CHS
_NGRPS = (_CHS + 15) // 16

_ROWS_PER_TILE = _N // _NS


def _silu(x):
    return x * jax.nn.sigmoid(x)


def _sc_gather(tbl, src, dst, off, e_seg):
    epw = e_seg // _NW
    nchunk = epw // _CH
    mesh = plsc.VectorSubcoreMesh(core_axis_name="c", subcore_axis_name="s")

    slot_types = [
        pltpu.VMEM((_CH,), jnp.int32),
        pltpu.VMEM((_CH,), jnp.int32),
        pltpu.VMEM((_CH, _D + _CP), jnp.float32),
        pltpu.VMEM((_CH, _D + _CP), jnp.float32),
        pltpu.VMEM((8, _CH), jnp.float32),
    ]

    @functools.partial(
        pl.kernel,
        out_type=(
            jax.ShapeDtypeStruct((e_seg, _D), jnp.float32),
            jax.ShapeDtypeStruct((e_seg, _D), jnp.float32),
            jax.ShapeDtypeStruct((8, e_seg), jnp.float32),
        ),
        mesh=mesh,
        scratch_types=slot_types + slot_types + [
            pltpu.SemaphoreType.DMA,
            pltpu.SemaphoreType.DMA,
            pltpu.SemaphoreType.DMA,
            pltpu.SemaphoreType.DMA,
        ],
        compiler_params=pltpu.CompilerParams(use_tc_tiling_on_sc=False,
                                             needs_layout_passes=False),
    )
    def k(tbl_hbm, src_hbm, dst_hbm,
          gs_hbm, gd_hbm, dT_hbm, *scr):
        slots = (tuple(scr[0:5]), tuple(scr[5:10]))
        semg = (scr[10], scr[11])
        sems = (scr[12], scr[13])
        c = lax.axis_index("c")
        s = lax.axis_index("s")
        wid = s * _NC + c
        base0 = off + wid * epw

        zero16 = jnp.zeros((16,), jnp.float32)
        for b in range(2):
            dT_b = slots[b][4]

            @pl.loop(0, _NGRP)
            def _(g, dT_b=dT_b):
                e0 = jnp.minimum(g * 16, _CH - 16)
                for r in range(4, 8):
                    dT_b[r, pl.ds(e0, 16)] = zero16

        def issue(j, b):
            idx_s, idx_d, buf_s, buf_d, _ = slots[b]
            base = base0 + j * _CH
            pltpu.sync_copy(src_hbm.at[pl.ds(base, _CH)], idx_s)
            pltpu.sync_copy(dst_hbm.at[pl.ds(base, _CH)], idx_d)
            pltpu.async_copy(tbl_hbm.at[idx_s], buf_s, semg[b])
            pltpu.async_copy(tbl_hbm.at[idx_d], buf_d, semg[b])

        def drain_stores(b):
            _, _, buf_s, buf_d, dT_buf = slots[b]
            pltpu.make_async_copy(buf_s.at[:, pl.ds(0, _D)],
                                  gs_hbm.at[pl.ds(0, _CH)], sems[b]).wait()
            pltpu.make_async_copy(buf_d.at[:, pl.ds(0, _D)],
                                  gd_hbm.at[pl.ds(0, _CH)], sems[b]).wait()
            pltpu.make_async_copy(dT_buf, dT_hbm.at[:, pl.ds(0, _CH)],
                                  sems[b]).wait()

        def process(j, b):
            idx_s, idx_d, buf_s, buf_d, dT_buf = slots[b]
            lbase = wid * epw + j * _CH
            pltpu.make_async_copy(tbl_hbm.at[idx_s], buf_s, semg[b]).wait()
            pltpu.make_async_copy(tbl_hbm.at[idx_d], buf_d, semg[b]).wait()

            @pl.loop(0, _NGRP)
            def _(g):
                e0 = jnp.minimum(g * 16, _CH - 16)
                rows = lax.iota(jnp.int32, 16) + e0
                d3 = []
                for cdim in range(3):
                    col = jnp.full((16,), _D + cdim, jnp.int32)
                    a = plsc.load_gather(buf_s, [rows, col])
                    b_ = plsc.load_gather(buf_d, [rows, col])
                    d3.append(a - b_)
                r2 = d3[0] * d3[0] + d3[1] * d3[1] + d3[2] * d3[2]
                for cdim in range(3):
                    dT_buf[cdim, pl.ds(e0, 16)] = d3[cdim]
                dT_buf[3, pl.ds(e0, 16)] = r2

            pltpu.async_copy(buf_s.at[:, pl.ds(0, _D)],
                             gs_hbm.at[pl.ds(lbase, _CH)], sems[b])
            pltpu.async_copy(buf_d.at[:, pl.ds(0, _D)],
                             gd_hbm.at[pl.ds(lbase, _CH)], sems[b])
            pltpu.async_copy(dT_buf, dT_hbm.at[:, pl.ds(lbase, _CH)], sems[b])

        issue(0, 0)

        @pl.loop(0, nchunk, step=2)
        def _(j):
            for b in range(2):
                jj = j + b
                nxt = jj + 1

                @pl.when(nxt < nchunk)
                def _(b=b, nxt=nxt):
                    @pl.when(nxt >= 2)
                    def _():
                        drain_stores(1 - b)
                    issue(nxt, 1 - b)

                @pl.when(jj < nchunk)
                def _(jj=jj, b=b):
                    process(jj, b)

        drain_stores(0)
        drain_stores(1)

    return k(tbl, src, dst)


_BE = 640


def _edge_body(nfs_ref, nfd_ref, dT_ref, ef_ref,
               Ws_ref, Wd_ref, Wec_ref, wr_ref, b1_ref,
               We2_ref, be2_ref, wa_ref, ba_ref, wco_ref,
               mh_ref, mxT_ref):
    tT = dT_ref[...].T
    r2 = tT[:, 3:4]
    radial = jnp.sqrt(r2 + 1e-12)

    pre = (jnp.dot(nfs_ref[...].astype(jnp.bfloat16), Ws_ref[...],
                   preferred_element_type=jnp.float32)
           + jnp.dot(nfd_ref[...].astype(jnp.bfloat16), Wd_ref[...],
                     preferred_element_type=jnp.float32)
           + jnp.dot(ef_ref[...].astype(jnp.bfloat16), Wec_ref[...],
                     preferred_element_type=jnp.float32)
           + radial * wr_ref[...]
           + b1_ref[...])
    h1 = _silu(pre[:, :_D])
    c1 = _silu(pre[:, _D:])
    m = _silu(jnp.dot(h1.astype(jnp.bfloat16), We2_ref[...],
                      preferred_element_type=jnp.float32)
              + be2_ref[...])
    attl = jnp.dot(m.astype(jnp.bfloat16), wa_ref[...],
                   preferred_element_type=jnp.float32)[:, :1]
    att = jax.nn.sigmoid(attl + ba_ref[...])
    mh_ref[...] = m * att
    cc = jnp.dot(c1.astype(jnp.bfloat16), wco_ref[...],
                 preferred_element_type=jnp.float32)[:, :1]
    gate = jnp.tanh(cc) * (_COORDS_RANGE / (radial + 1.0))
    lane8 = lax.broadcasted_iota(jnp.int32, (1, 8), 1)
    mxT = (tT * gate * (lane8 < 3)).T
    mxT_ref[...] = mxT


def _tc_edge(nfs, nfd, dT, ef, off, Ws, Wd, Wec, wr, b1, We2, be2, wa, ba, wco):
    e_seg = nfs.shape[0]
    nblk = e_seg // _BE
    off_blk = off // _BE
    full = lambda r, c_: pl.BlockSpec((r, c_), lambda i: (0, 0))
    blk = lambda c_: pl.BlockSpec((_BE, c_), lambda i: (i, 0))
    efblk = pl.BlockSpec((_BE, 16), lambda i: (i + off_blk, 0))
    tblk = pl.BlockSpec((8, _BE), lambda i: (0, i))
    return pl.pallas_call(
        _edge_body,
        grid=(nblk,),
        in_specs=[
            blk(_D), blk(_D), tblk, efblk,
            full(_D, 256), full(_D, 256), full(16, 256), full(1, 256),
            full(1, 256), full(_D, _D), full(1, _D), full(_D, 8),
            full(1, 1), full(_D, 8),
        ],
        out_specs=[blk(_D), tblk],
        out_shape=(
            jax.ShapeDtypeStruct((e_seg, _D), jnp.float32),
            jax.ShapeDtypeStruct((8, e_seg), jnp.float32),
        ),
    )(nfs, nfd, dT, ef, Ws, Wd, Wec, wr, b1, We2, be2, wa, ba, wco)


def _sc_scatter(mh, mxT, dst, off, zh, zx):
    e_seg = mh.shape[0]
    epw = e_seg // _NW
    nchunks = epw // _CHS
    mesh = plsc.VectorSubcoreMesh(core_axis_name="c", subcore_axis_name="s")

    @functools.partial(
        pl.kernel,
        out_type=(
            jax.ShapeDtypeStruct((_NC, _N, _D), jnp.float32),
            jax.ShapeDtypeStruct((_NC, _N, _CP), jnp.float32),
        ),
        mesh=mesh,
        scratch_types=[
            pltpu.VMEM_SHARED((_N, _D), jnp.float32),
            pltpu.VMEM_SHARED((_N, _CP), jnp.float32),
            pltpu.VMEM((_CHS,), jnp.int32),
            pltpu.VMEM((_CHS, _D), jnp.float32),
            pltpu.VMEM((8, _CHS), jnp.float32),
            pltpu.VMEM((_CHS, _CP), jnp.float32),
            pltpu.SemaphoreType.DMA,
        ],
        compiler_params=pltpu.CompilerParams(use_tc_tiling_on_sc=False,
                                             needs_layout_passes=False),
    )
    def k(mh_hbm, mxT_hbm, dst_hbm, zh_hbm, zx_hbm,
          ph_hbm, px_hbm,
          h_acc, x_acc, idx_v, buf_h, bufT, buf_x, sem):
        c = lax.axis_index("c")
        s = lax.axis_index("s")
        wid = s * _NC + c
        base0 = wid * epw
        row0 = s * _ROWS_PER_TILE
        gbase0 = off + base0

        zero16 = jnp.zeros((16,), jnp.float32)

        @pl.loop(0, _CHS)
        def _(r):
            buf_x[r] = zero16

        pltpu.sync_copy(zh_hbm.at[pl.ds(row0, _ROWS_PER_TILE)],
                        h_acc.at[pl.ds(row0, _ROWS_PER_TILE)])
        pltpu.sync_copy(zx_hbm.at[pl.ds(row0, _ROWS_PER_TILE)],
                        x_acc.at[pl.ds(row0, _ROWS_PER_TILE)])
        plsc.subcore_barrier()

        @pl.loop(0, nchunks)
        def _(j):
            gbase = gbase0 + j * _CHS
            base = base0 + j * _CHS
            pltpu.async_copy(dst_hbm.at[pl.ds(gbase, _CHS)], idx_v, sem)
            pltpu.async_copy(mh_hbm.at[pl.ds(base, _CHS)], buf_h, sem)
            pltpu.async_copy(mxT_hbm.at[:, pl.ds(base, _CHS)], bufT, sem)
            pltpu.make_async_copy(dst_hbm.at[pl.ds(gbase, _CHS)], idx_v,
                                  sem).wait()
            pltpu.make_async_copy(mh_hbm.at[pl.ds(base, _CHS)], buf_h,
                                  sem).wait()
            pltpu.make_async_copy(mxT_hbm.at[:, pl.ds(base, _CHS)], bufT,
                                  sem).wait()

            @pl.loop(0, _NGRPS)
            def _(g):
                e0 = jnp.minimum(g * 16, _CHS - 16)
                rows = lax.iota(jnp.int32, 16) + e0
                for cdim in range(3):
                    col = jnp.full((16,), cdim, jnp.int32)
                    vec = bufT[cdim, pl.ds(e0, 16)]
                    plsc.store_scatter(buf_x, [rows, col], vec)

            pltpu.sync_copy(buf_h, h_acc.at[idx_v], add=True)
            pltpu.sync_copy(buf_x, x_acc.at[idx_v], add=True)

        plsc.subcore_barrier()

        pltpu.sync_copy(h_acc.at[pl.ds(row0, _ROWS_PER_TILE)],
                        ph_hbm.at[c].at[pl.ds(row0, _ROWS_PER_TILE)])
        pltpu.sync_copy(x_acc.at[pl.ds(row0, _ROWS_PER_TILE)],
                        px_hbm.at[c].at[pl.ds(row0, _ROWS_PER_TILE)])

    return k(mh, mxT, dst, zh, zx)


_BN = 1000


def _node_body(nf_ref, c16_ref, z_ref,
               ph0_ref, ph1_ref, ph2_ref, ph3_ref,
               px0_ref, px1_ref, px2_ref, px3_ref,
               Wn1a_ref, Wn1b_ref, bn1_ref, Wn2_ref, bn2_ref, g_ref, b_ref,
               h_ref, x_ref):
    zinv = 1.0 / z_ref[...]
    hn = ((ph0_ref[...] + ph1_ref[...])
          + (ph2_ref[...] + ph3_ref[...])) * zinv
    xn = ((px0_ref[...] + px1_ref[...])
          + (px2_ref[...] + px3_ref[...])) * zinv
    t = _silu(jnp.dot(nf_ref[...].astype(jnp.bfloat16), Wn1a_ref[...],
                      preferred_element_type=jnp.float32)
              + jnp.dot(hn.astype(jnp.bfloat16), Wn1b_ref[...],
                        preferred_element_type=jnp.float32)
              + bn1_ref[...])
    h = jnp.dot(t.astype(jnp.bfloat16), Wn2_ref[...],
                preferred_element_type=jnp.float32) + bn2_ref[...]
    mu = jnp.mean(h, axis=1, keepdims=True)
    var = jnp.mean((h - mu) * (h - mu), axis=1, keepdims=True)
    h_ref[...] = (h - mu) / jnp.sqrt(var + 1e-5) * g_ref[...] + b_ref[...]
    x_ref[...] = c16_ref[...] + xn


def _tc_node(nf, c16, z, phs, pxs, Wn1a, Wn1b, bn1, Wn2, bn2, g, b):
    nblk = _N // _BN
    full = lambda r, c_: pl.BlockSpec((r, c_), lambda i: (0, 0))
    blk = lambda c_: pl.BlockSpec((_BN, c_), lambda i: (i, 0))
    return pl.pallas_call(
        _node_body,
        grid=(nblk,),
        in_specs=[
            blk(_D), blk(_CP), blk(1),
            blk(_D), blk(_D), blk(_D), blk(_D),
            blk(_CP), blk(_CP), blk(_CP), blk(_CP),
            full(_D, _D), full(_D, _D), full(1, _D), full(_D, _D),
            full(1, _D), full(1, _D), full(1, _D),
        ],
        out_specs=[blk(_D), blk(_CP)],
        out_shape=(
            jax.ShapeDtypeStruct((_N, _D), jnp.float32),
            jax.ShapeDtypeStruct((_N, _CP), jnp.float32),
        ),
    )(nf, c16, z, *phs, *pxs, Wn1a, Wn1b, bn1, Wn2, bn2, g, b)


def kernel(node_feat, coord_feat, z, edge_feat, edge_index,
           We1, be1, We2, be2, Wa, ba, Wc1, bc1, Wc_out,
           Wn1, bn1, Wn2, bn2, ln_g, ln_b):
    src = edge_index[0].astype(jnp.int32)
    dst = edge_index[1].astype(jnp.int32)
    c16 = jnp.pad(coord_feat, ((0, 0), (0, _CP - 3)))

    Ws = jnp.concatenate([We1[:_D], Wc1[:_D]], axis=1).astype(jnp.bfloat16)
    Wd = jnp.concatenate([We1[_D:2 * _D], Wc1[_D:2 * _D]], axis=1).astype(jnp.bfloat16)
    Wec = jnp.concatenate([We1[2 * _D + 1:], Wc1[2 * _D + 1:]],
                          axis=1).astype(jnp.bfloat16)
    wr = jnp.concatenate([We1[2 * _D], Wc1[2 * _D]])[None, :]
    b1 = jnp.concatenate([be1, bc1])[None, :]
    be2r = be2[None, :]
    wa = jnp.tile(Wa, (1, 8)).astype(jnp.bfloat16)
    bar = ba.reshape(1, 1)
    wco = jnp.tile(Wc_out, (1, 8)).astype(jnp.bfloat16)
    Wn1a = Wn1[:_D].astype(jnp.bfloat16)
    Wn1b = Wn1[_D:].astype(jnp.bfloat16)
    Wn2b = Wn2.astype(jnp.bfloat16)
    bn1r = bn1[None, :]
    bn2r = bn2[None, :]
    gr = ln_g[None, :]
    br = ln_b[None, :]

    zh = jnp.zeros((_N, _D), jnp.float32)
    zx = jnp.zeros((_N, _CP), jnp.float32)
    We2b = We2.astype(jnp.bfloat16)

    half = _E // 2
    phs, pxs = [], []
    tbl = jnp.concatenate([node_feat, c16], axis=1)
    gathered = [_sc_gather(tbl, src, dst, lo, half) for lo in (0, half)]
    for (lo, (gs, gd, dT)) in zip((0, half), gathered):
        mh, mxT = _tc_edge(gs, gd, dT, edge_feat, lo,
                           Ws, Wd, Wec, wr, b1, We2b,
                           be2r, wa, bar, wco)
        ph, px = _sc_scatter(mh, mxT, dst, lo, zh, zx)
        phs.extend([ph[0], ph[1]])
        pxs.extend([px[0], px[1]])
    h, x16 = _tc_node(node_feat, c16, z, phs, pxs,
                      Wn1a, Wn1b, bn1r, Wn2b, bn2r, gr, br)
    return (h, x16[:, :3])


# speedup vs baseline: 1.0106x; 1.0106x over previous
import functools

import jax
import jax.numpy as jnp
from jax import lax
from jax.experimental import pallas as pl
from jax.experimental.pallas import tpu as pltpu
from jax.experimental.pallas import tpu_sc as plsc

_N = 10000
_E = 320000
_D = 128
_CP = 16
_COORDS_RANGE = 10.0

_NC = 2
_NS = 16
_NW = _NC * _NS
_EPW = _E // _NW

_CH = 200
_NCHUNK = _EPW // _CH
_NGRP = (_CH + 15) // 16

_CHS = 200
_NCHUNKS = _EPW // _CHS
_NGRPS = (_CHS + 15) // 16

_ROWS_PER_TILE = _N // _NS


def _silu(x):
    return x * jax.nn.sigmoid(x)


def _sc_gather(nf, c16, src, dst, off, e_seg):
    epw = e_seg // _NW
    nchunk = epw // _CH
    mesh = plsc.VectorSubcoreMesh(core_axis_name="c", subcore_axis_name="s")

    slot_types = [
        pltpu.VMEM((_CH,), jnp.int32),
        pltpu.VMEM((_CH,), jnp.int32),
        pltpu.VMEM((_CH, _D), jnp.float32),
        pltpu.VMEM((_CH, _D), jnp.float32),
        pltpu.VMEM((_CH, _CP), jnp.float32),
        pltpu.VMEM((_CH, _CP), jnp.float32),
        pltpu.VMEM((8, _CH), jnp.float32),
    ]

    @functools.partial(
        pl.kernel,
        out_type=(
            jax.ShapeDtypeStruct((e_seg, _D), jnp.float32),
            jax.ShapeDtypeStruct((e_seg, _D), jnp.float32),
            jax.ShapeDtypeStruct((8, e_seg), jnp.float32),
        ),
        mesh=mesh,
        scratch_types=slot_types + slot_types + [
            pltpu.SemaphoreType.DMA,
            pltpu.SemaphoreType.DMA,
            pltpu.SemaphoreType.DMA,
            pltpu.SemaphoreType.DMA,
        ],
        compiler_params=pltpu.CompilerParams(use_tc_tiling_on_sc=False,
                                             needs_layout_passes=False),
    )
    def k(nf_hbm, c16_hbm, src_hbm, dst_hbm,
          gs_hbm, gd_hbm, dT_hbm, *scr):
        slots = (tuple(scr[0:7]), tuple(scr[7:14]))
        semg = (scr[14], scr[15])
        sems = (scr[16], scr[17])
        c = lax.axis_index("c")
        s = lax.axis_index("s")
        wid = s * _NC + c
        base0 = off + wid * epw

        zero16 = jnp.zeros((16,), jnp.float32)
        for b in range(2):
            dT_b = slots[b][6]

            @pl.loop(0, _NGRP)
            def _(g, dT_b=dT_b):
                e0 = jnp.minimum(g * 16, _CH - 16)
                for r in range(4, 8):
                    dT_b[r, pl.ds(e0, 16)] = zero16

        def issue(j, b):
            idx_s, idx_d, buf_s, buf_d, buf_cs, buf_cd, _ = slots[b]
            base = base0 + j * _CH
            pltpu.sync_copy(src_hbm.at[pl.ds(base, _CH)], idx_s)
            pltpu.sync_copy(dst_hbm.at[pl.ds(base, _CH)], idx_d)
            pltpu.async_copy(nf_hbm.at[idx_s], buf_s, semg[b])
            pltpu.async_copy(nf_hbm.at[idx_d], buf_d, semg[b])
            pltpu.async_copy(c16_hbm.at[idx_s], buf_cs, semg[b])
            pltpu.async_copy(c16_hbm.at[idx_d], buf_cd, semg[b])

        def drain_stores(b):
            _, _, buf_s, buf_d, _, _, dT_buf = slots[b]
            pltpu.make_async_copy(buf_s, gs_hbm.at[pl.ds(0, _CH)],
                                  sems[b]).wait()
            pltpu.make_async_copy(buf_d, gd_hbm.at[pl.ds(0, _CH)],
                                  sems[b]).wait()
            pltpu.make_async_copy(dT_buf, dT_hbm.at[:, pl.ds(0, _CH)],
                                  sems[b]).wait()

        def process(j, b):
            idx_s, idx_d, buf_s, buf_d, buf_cs, buf_cd, dT_buf = slots[b]
            lbase = wid * epw + j * _CH
            pltpu.make_async_copy(nf_hbm.at[idx_s], buf_s, semg[b]).wait()
            pltpu.make_async_copy(nf_hbm.at[idx_d], buf_d, semg[b]).wait()
            pltpu.make_async_copy(c16_hbm.at[idx_s], buf_cs, semg[b]).wait()
            pltpu.make_async_copy(c16_hbm.at[idx_d], buf_cd, semg[b]).wait()

            @pl.loop(0, _NGRP)
            def _(g):
                e0 = jnp.minimum(g * 16, _CH - 16)
                rows = lax.iota(jnp.int32, 16) + e0
                d3 = []
                for cdim in range(3):
                    col = jnp.full((16,), cdim, jnp.int32)
                    a = plsc.load_gather(buf_cs, [rows, col])
                    b_ = plsc.load_gather(buf_cd, [rows, col])
                    d3.append(a - b_)
                r2 = d3[0] * d3[0] + d3[1] * d3[1] + d3[2] * d3[2]
                for cdim in range(3):
                    dT_buf[cdim, pl.ds(e0, 16)] = d3[cdim]
                dT_buf[3, pl.ds(e0, 16)] = r2

            pltpu.async_copy(buf_s, gs_hbm.at[pl.ds(lbase, _CH)], sems[b])
            pltpu.async_copy(buf_d, gd_hbm.at[pl.ds(lbase, _CH)], sems[b])
            pltpu.async_copy(dT_buf, dT_hbm.at[:, pl.ds(lbase, _CH)], sems[b])

        issue(0, 0)

        @pl.loop(0, nchunk, step=2)
        def _(j):
            for b in range(2):
                jj = j + b
                nxt = jj + 1

                @pl.when(nxt < nchunk)
                def _(b=b, nxt=nxt):
                    @pl.when(nxt >= 2)
                    def _():
                        drain_stores(1 - b)
                    issue(nxt, 1 - b)

                @pl.when(jj < nchunk)
                def _(jj=jj, b=b):
                    process(jj, b)

        drain_stores(0)
        drain_stores(1)

    return k(nf, c16, src, dst)


_BE = 640


def _edge_body(nfs_ref, nfd_ref, dT_ref, ef_ref,
               Ws_ref, Wd_ref, Wec_ref, wr_ref, b1_ref,
               We2_ref, be2_ref, wa_ref, ba_ref, wco_ref,
               mh_ref, mxT_ref):
    tT = dT_ref[...].T
    r2 = tT[:, 3:4]
    radial = jnp.sqrt(r2 + 1e-12)

    pre = (jnp.dot(nfs_ref[...].astype(jnp.bfloat16), Ws_ref[...],
                   preferred_element_type=jnp.float32)
           + jnp.dot(nfd_ref[...].astype(jnp.bfloat16), Wd_ref[...],
                     preferred_element_type=jnp.float32)
           + jnp.dot(ef_ref[...].astype(jnp.bfloat16), Wec_ref[...],
                     preferred_element_type=jnp.float32)
           + radial * wr_ref[...]
           + b1_ref[...])
    h1 = _silu(pre[:, :_D])
    c1 = _silu(pre[:, _D:])
    m = _silu(jnp.dot(h1.astype(jnp.bfloat16), We2_ref[...],
                      preferred_element_type=jnp.float32)
              + be2_ref[...])
    attl = jnp.dot(m.astype(jnp.bfloat16), wa_ref[...],
                   preferred_element_type=jnp.float32)[:, :1]
    att = jax.nn.sigmoid(attl + ba_ref[...])
    mh_ref[...] = m * att
    cc = jnp.dot(c1.astype(jnp.bfloat16), wco_ref[...],
                 preferred_element_type=jnp.float32)[:, :1]
    gate = jnp.tanh(cc) * (_COORDS_RANGE / (radial + 1.0))
    lane8 = lax.broadcasted_iota(jnp.int32, (1, 8), 1)
    mxT = (tT * gate * (lane8 < 3)).T
    mxT_ref[...] = mxT


def _tc_edge(nfs, nfd, dT, ef, off, Ws, Wd, Wec, wr, b1, We2, be2, wa, ba, wco):
    e_seg = nfs.shape[0]
    nblk = e_seg // _BE
    off_blk = off // _BE
    full = lambda r, c_: pl.BlockSpec((r, c_), lambda i: (0, 0))
    blk = lambda c_: pl.BlockSpec((_BE, c_), lambda i: (i, 0))
    efblk = pl.BlockSpec((_BE, 16), lambda i: (i + off_blk, 0))
    tblk = pl.BlockSpec((8, _BE), lambda i: (0, i))
    return pl.pallas_call(
        _edge_body,
        grid=(nblk,),
        in_specs=[
            blk(_D), blk(_D), tblk, efblk,
            full(_D, 256), full(_D, 256), full(16, 256), full(1, 256),
            full(1, 256), full(_D, _D), full(1, _D), full(_D, 8),
            full(1, 1), full(_D, 8),
        ],
        out_specs=[blk(_D), tblk],
        out_shape=(
            jax.ShapeDtypeStruct((e_seg, _D), jnp.float32),
            jax.ShapeDtypeStruct((8, e_seg), jnp.float32),
        ),
    )(nfs, nfd, dT, ef, Ws, Wd, Wec, wr, b1, We2, be2, wa, ba, wco)


def _sc_scatter(mh, mxT, dst, off, zh, zx):
    e_seg = mh.shape[0]
    epw = e_seg // _NW
    nchunks = epw // _CHS
    mesh = plsc.VectorSubcoreMesh(core_axis_name="c", subcore_axis_name="s")

    @functools.partial(
        pl.kernel,
        out_type=(
            jax.ShapeDtypeStruct((_NC, _N, _D), jnp.float32),
            jax.ShapeDtypeStruct((_NC, _N, _CP), jnp.float32),
        ),
        mesh=mesh,
        scratch_types=[
            pltpu.VMEM_SHARED((_N, _D), jnp.float32),
            pltpu.VMEM_SHARED((_N, _CP), jnp.float32),
            pltpu.VMEM((_CHS,), jnp.int32),
            pltpu.VMEM((_CHS, _D), jnp.float32),
            pltpu.VMEM((8, _CHS), jnp.float32),
            pltpu.VMEM((_CHS, _CP), jnp.float32),
            pltpu.SemaphoreType.DMA,
        ],
        compiler_params=pltpu.CompilerParams(use_tc_tiling_on_sc=False,
                                             needs_layout_passes=False),
    )
    def k(mh_hbm, mxT_hbm, dst_hbm, zh_hbm, zx_hbm,
          ph_hbm, px_hbm,
          h_acc, x_acc, idx_v, buf_h, bufT, buf_x, sem):
        c = lax.axis_index("c")
        s = lax.axis_index("s")
        wid = s * _NC + c
        base0 = wid * epw
        row0 = s * _ROWS_PER_TILE
        gbase0 = off + base0

        zero16 = jnp.zeros((16,), jnp.float32)

        @pl.loop(0, _CHS)
        def _(r):
            buf_x[r] = zero16

        pltpu.sync_copy(zh_hbm.at[pl.ds(row0, _ROWS_PER_TILE)],
                        h_acc.at[pl.ds(row0, _ROWS_PER_TILE)])
        pltpu.sync_copy(zx_hbm.at[pl.ds(row0, _ROWS_PER_TILE)],
                        x_acc.at[pl.ds(row0, _ROWS_PER_TILE)])
        plsc.subcore_barrier()

        @pl.loop(0, nchunks)
        def _(j):
            gbase = gbase0 + j * _CHS
            base = base0 + j * _CHS
            pltpu.async_copy(dst_hbm.at[pl.ds(gbase, _CHS)], idx_v, sem)
            pltpu.async_copy(mh_hbm.at[pl.ds(base, _CHS)], buf_h, sem)
            pltpu.async_copy(mxT_hbm.at[:, pl.ds(base, _CHS)], bufT, sem)
            pltpu.make_async_copy(dst_hbm.at[pl.ds(gbase, _CHS)], idx_v,
                                  sem).wait()
            pltpu.make_async_copy(mh_hbm.at[pl.ds(base, _CHS)], buf_h,
                                  sem).wait()
            pltpu.make_async_copy(mxT_hbm.at[:, pl.ds(base, _CHS)], bufT,
                                  sem).wait()

            @pl.loop(0, _NGRPS)
            def _(g):
                e0 = jnp.minimum(g * 16, _CHS - 16)
                rows = lax.iota(jnp.int32, 16) + e0
                for cdim in range(3):
                    col = jnp.full((16,), cdim, jnp.int32)
                    vec = bufT[cdim, pl.ds(e0, 16)]
                    plsc.store_scatter(buf_x, [rows, col], vec)

            pltpu.sync_copy(buf_h, h_acc.at[idx_v], add=True)
            pltpu.sync_copy(buf_x, x_acc.at[idx_v], add=True)

        plsc.subcore_barrier()

        pltpu.sync_copy(h_acc.at[pl.ds(row0, _ROWS_PER_TILE)],
                        ph_hbm.at[c].at[pl.ds(row0, _ROWS_PER_TILE)])
        pltpu.sync_copy(x_acc.at[pl.ds(row0, _ROWS_PER_TILE)],
                        px_hbm.at[c].at[pl.ds(row0, _ROWS_PER_TILE)])

    return k(mh, mxT, dst, zh, zx)


_BN = 1000


def _node_body(nf_ref, c16_ref, z_ref,
               ph0_ref, ph1_ref, ph2_ref, ph3_ref,
               px0_ref, px1_ref, px2_ref, px3_ref,
               Wn1a_ref, Wn1b_ref, bn1_ref, Wn2_ref, bn2_ref, g_ref, b_ref,
               h_ref, x_ref):
    zinv = 1.0 / z_ref[...]
    hn = ((ph0_ref[...] + ph1_ref[...])
          + (ph2_ref[...] + ph3_ref[...])) * zinv
    xn = ((px0_ref[...] + px1_ref[...])
          + (px2_ref[...] + px3_ref[...])) * zinv
    t = _silu(jnp.dot(nf_ref[...].astype(jnp.bfloat16), Wn1a_ref[...],
                      preferred_element_type=jnp.float32)
              + jnp.dot(hn.astype(jnp.bfloat16), Wn1b_ref[...],
                        preferred_element_type=jnp.float32)
              + bn1_ref[...])
    h = jnp.dot(t.astype(jnp.bfloat16), Wn2_ref[...],
                preferred_element_type=jnp.float32) + bn2_ref[...]
    mu = jnp.mean(h, axis=1, keepdims=True)
    var = jnp.mean((h - mu) * (h - mu), axis=1, keepdims=True)
    h_ref[...] = (h - mu) / jnp.sqrt(var + 1e-5) * g_ref[...] + b_ref[...]
    x_ref[...] = c16_ref[...] + xn


def _tc_node(nf, c16, z, phs, pxs, Wn1a, Wn1b, bn1, Wn2, bn2, g, b):
    nblk = _N // _BN
    full = lambda r, c_: pl.BlockSpec((r, c_), lambda i: (0, 0))
    blk = lambda c_: pl.BlockSpec((_BN, c_), lambda i: (i, 0))
    return pl.pallas_call(
        _node_body,
        grid=(nblk,),
        in_specs=[
            blk(_D), blk(_CP), blk(1),
            blk(_D), blk(_D), blk(_D), blk(_D),
            blk(_CP), blk(_CP), blk(_CP), blk(_CP),
            full(_D, _D), full(_D, _D), full(1, _D), full(_D, _D),
            full(1, _D), full(1, _D), full(1, _D),
        ],
        out_specs=[blk(_D), blk(_CP)],
        out_shape=(
            jax.ShapeDtypeStruct((_N, _D), jnp.float32),
            jax.ShapeDtypeStruct((_N, _CP), jnp.float32),
        ),
    )(nf, c16, z, *phs, *pxs, Wn1a, Wn1b, bn1, Wn2, bn2, g, b)


def kernel(node_feat, coord_feat, z, edge_feat, edge_index,
           We1, be1, We2, be2, Wa, ba, Wc1, bc1, Wc_out,
           Wn1, bn1, Wn2, bn2, ln_g, ln_b):
    src = edge_index[0].astype(jnp.int32)
    dst = edge_index[1].astype(jnp.int32)
    c16 = jnp.pad(coord_feat, ((0, 0), (0, _CP - 3)))

    Ws = jnp.concatenate([We1[:_D], Wc1[:_D]], axis=1).astype(jnp.bfloat16)
    Wd = jnp.concatenate([We1[_D:2 * _D], Wc1[_D:2 * _D]], axis=1).astype(jnp.bfloat16)
    Wec = jnp.concatenate([We1[2 * _D + 1:], Wc1[2 * _D + 1:]],
                          axis=1).astype(jnp.bfloat16)
    wr = jnp.concatenate([We1[2 * _D], Wc1[2 * _D]])[None, :]
    b1 = jnp.concatenate([be1, bc1])[None, :]
    be2r = be2[None, :]
    wa = jnp.tile(Wa, (1, 8)).astype(jnp.bfloat16)
    bar = ba.reshape(1, 1)
    wco = jnp.tile(Wc_out, (1, 8)).astype(jnp.bfloat16)
    Wn1a = Wn1[:_D].astype(jnp.bfloat16)
    Wn1b = Wn1[_D:].astype(jnp.bfloat16)
    Wn2b = Wn2.astype(jnp.bfloat16)
    bn1r = bn1[None, :]
    bn2r = bn2[None, :]
    gr = ln_g[None, :]
    br = ln_b[None, :]

    zh = jnp.zeros((_N, _D), jnp.float32)
    zx = jnp.zeros((_N, _CP), jnp.float32)
    We2b = We2.astype(jnp.bfloat16)

    half = _E // 2
    phs, pxs = [], []
    gathered = [_sc_gather(node_feat, c16, src, dst, lo, half)
                for lo in (0, half)]
    for (lo, (gs, gd, dT)) in zip((0, half), gathered):
        mh, mxT = _tc_edge(gs, gd, dT, edge_feat, lo,
                           Ws, Wd, Wec, wr, b1, We2b,
                           be2r, wa, bar, wco)
        ph, px = _sc_scatter(mh, mxT, dst, lo, zh, zx)
        phs.extend([ph[0], ph[1]])
        pxs.extend([px[0], px[1]])
    h, x16 = _tc_node(node_feat, c16, z, phs, pxs,
                      Wn1a, Wn1b, bn1r, Wn2b, bn2r, gr, br)
    return (h, x16[:, :3])
